# baseline (speedup 1.0000x reference)
import functools

import jax
import jax.numpy as jnp
from jax import lax
from jax.experimental import pallas as pl
from jax.experimental.pallas import tpu as pltpu
from jax.experimental.pallas import tpu_sc as plsc

N = 10000
E = 160000
D_IN = 256
D_HID = 256
D_OUT = 128

NC = 2
NS = 16
CH = 128
EPAD = 163840
EROWS = EPAD // CH
NROWS = 10016
RPT = NROWS // NS
NBUF = 8
LEAD = 4

_mesh = plsc.VectorSubcoreMesh(core_axis_name="c", subcore_axis_name="s")
_params = pltpu.CompilerParams(use_tc_tiling_on_sc=False)


@functools.partial(
    pl.kernel,
    out_type=jax.ShapeDtypeStruct((2 * NROWS, 16), jnp.int32),
    mesh=_mesh,
    compiler_params=_params,
    scratch_types=[
        pltpu.VMEM((EROWS // 32, CH), jnp.int32),
        pltpu.VMEM((CH, 16), jnp.int32),
        pltpu.MemorySpace.VMEM_SHARED((NROWS, 16), jnp.int32),
        pltpu.SemaphoreType.DMA,
    ],
)
def _sc_hist(dst_hbm, ones_hbm, zero_hbm, out_hbm, dst_v, ones_v, acc, sem):
    c = lax.axis_index("c")
    s = lax.axis_index("s")
    nrow = EROWS // 32
    pltpu.sync_copy(zero_hbm.at[pl.ds(s * RPT, RPT)],
                    acc.at[pl.ds(s * RPT, RPT)])
    pltpu.sync_copy(ones_hbm, ones_v)
    pltpu.sync_copy(dst_hbm.at[pl.ds((c * NS + s) * nrow, nrow)], dst_v)
    plsc.subcore_barrier()

    def body(j, _):
        pltpu.async_copy(ones_v, acc.at[dst_v.at[j]], sem, add=True)
        return _

    lax.fori_loop(0, nrow, body, None)

    def drain(j, _):
        pltpu.make_async_copy(ones_v, acc.at[dst_v.at[j]], sem).wait()
        return _

    lax.fori_loop(0, nrow, drain, None)
    plsc.subcore_barrier()
    pltpu.sync_copy(acc.at[pl.ds(s * RPT, RPT)],
                    out_hbm.at[pl.ds(c * NROWS + s * RPT, RPT)])


F = 64


def _make_scat(nphase):
    nrow = EROWS // NS

    @functools.partial(
        pl.kernel,
        out_type=jax.ShapeDtypeStruct((2 * nphase * NROWS, F), jnp.float32),
        mesh=_mesh,
        compiler_params=_params,
        scratch_types=(
            [
                pltpu.VMEM((nrow, CH), jnp.int32),
                pltpu.VMEM((nrow, CH), jnp.int32),
                pltpu.VMEM((NBUF, CH, F), jnp.float32),
                pltpu.MemorySpace.VMEM_SHARED((NROWS, F), jnp.float32),
            ]
            + [pltpu.SemaphoreType.DMA] * (2 * NBUF)
        ),
    )
    def scat(tab_hbm, srcs_hbm, dst_hbm, zero_hbm, out_hbm,
             src_v, dst_v, rows, acc, *sems):
        gsem = sems[:NBUF]
        ssem = sems[NBUF:]
        c = lax.axis_index("c")
        s = lax.axis_index("s")
        pltpu.sync_copy(dst_hbm.at[pl.ds(s * nrow, nrow)], dst_v)
        for p in range(nphase):
            pltpu.sync_copy(
                srcs_hbm.at[pl.ds((p * 2 + c) * EROWS + s * nrow, nrow)],
                src_v)
            for b in range(LEAD):
                pltpu.async_copy(tab_hbm.at[src_v.at[b]], rows.at[b], gsem[b])
            pltpu.sync_copy(zero_hbm.at[pl.ds(s * RPT, RPT)],
                            acc.at[pl.ds(s * RPT, RPT)])
            plsc.subcore_barrier()

            def body(jj, _):
                for b in range(NBUF):
                    j = jj * NBUF + b
                    pltpu.make_async_copy(tab_hbm.at[src_v.at[j]],
                                          rows.at[b], gsem[b]).wait()
                    pltpu.async_copy(rows.at[b], acc.at[dst_v.at[j]],
                                     ssem[b], add=True)
                    pj = j + LEAD
                    pb = (b + LEAD) % NBUF

                    @pl.when(pj < nrow)
                    def _():
                        @pl.when(pj >= NBUF)
                        def _():
                            pltpu.make_async_copy(
                                rows.at[pb], acc.at[dst_v.at[pj - NBUF]],
                                ssem[pb]).wait()

                        pltpu.async_copy(tab_hbm.at[src_v.at[pj]],
                                         rows.at[pb], gsem[pb])
                return _

            lax.fori_loop(0, nrow // NBUF, body, None)
            for b in range(NBUF):
                j = nrow - NBUF + b
                pltpu.make_async_copy(rows.at[b], acc.at[dst_v.at[j]],
                                      ssem[b]).wait()
            plsc.subcore_barrier()
            pltpu.sync_copy(
                acc.at[pl.ds(s * RPT, RPT)],
                out_hbm.at[pl.ds((p * 2 + c) * NROWS + s * RPT, RPT)])
            if p + 1 < nphase:
                plsc.subcore_barrier()

    return scat


_sc_scat1 = _make_scat(2)
_sc_scat2 = _make_scat(1)


R = 1000
G = N // R


def _cnt_block(hist_ref):
    return (hist_ref[0] + hist_ref[1]).astype(jnp.float32)


def _tc_pre(x_ref, w_ref, hist_ref, out_ref):
    h = jnp.dot(x_ref[...], w_ref[...], preferred_element_type=jnp.float32)
    h = h * lax.rsqrt(_cnt_block(hist_ref) + 1.0)
    for q in range(4):
        out_ref[q] = h[:, q * F:(q + 1) * F]


def _tc_mid(agg_ref, h2_ref, hist_ref, b1_ref, lnw_ref, lnb_ref,
            wrel_ref, wroot_ref, hrs_ref, hroot_ref, mom_ref, g_buf):
    pid = pl.program_id(0)

    @pl.when(pid < G)
    def _():
        dinv = lax.rsqrt(_cnt_block(hist_ref) + 1.0)
        aggc = jnp.concatenate([agg_ref[0], agg_ref[1],
                                agg_ref[2], agg_ref[3]], axis=1)
        h2c = jnp.concatenate([h2_ref[0], h2_ref[1], h2_ref[2], h2_ref[3]],
                              axis=1)
        g = (aggc + h2c) * dinv + b1_ref[...]
        g_buf[pl.ds(pid * R, R), :] = g
        srow = lax.broadcasted_iota(jnp.int32, (8, 128), 0)
        scol = lax.broadcasted_iota(jnp.int32, (8, 128), 1)
        contrib = (jnp.where((srow == 0) & (scol == 0), jnp.sum(g), 0.0)
                   + jnp.where((srow == 0) & (scol == 1), jnp.sum(g * g),
                               0.0))

        @pl.when(pid == 0)
        def _():
            mom_ref[...] = contrib

        @pl.when(pid > 0)
        def _():
            mom_ref[...] = mom_ref[...] + contrib

    @pl.when(pid >= G)
    def _():
        tot = float(N * D_HID)
        m = mom_ref[...]
        mean = m[0, 0] / tot
        var = m[0, 1] / tot - mean * mean
        std = jnp.sqrt(jnp.maximum(var, 0.0))
        gblk = g_buf[pl.ds((pid - G) * R, R), :]
        hh = (gblk - mean) / (std + 1e-5) * lnw_ref[...] + lnb_ref[...]
        hh = jnp.maximum(hh, 0.0)
        hr = jnp.dot(hh, wrel_ref[...], preferred_element_type=jnp.float32)
        hroot_ref[...] = jnp.dot(hh, wroot_ref[...],
                                 preferred_element_type=jnp.float32)
        hrs_ref[0] = hr[:, :64]
        hrs_ref[1] = hr[:, 64:]


def _tc_post(agg_ref, hroot_ref, hist_ref, brel_ref, emb_ref, logp_ref):
    cnt = _cnt_block(hist_ref)
    aggc = jnp.concatenate([agg_ref[0], agg_ref[1]], axis=1)
    emb = aggc / jnp.maximum(cnt, 1.0) + brel_ref[...] + hroot_ref[...]
    emb_ref[...] = emb
    sh = emb - jnp.max(emb, axis=1, keepdims=True)
    logp_ref[...] = sh - jnp.log(jnp.sum(jnp.exp(sh), axis=1, keepdims=True))


def kernel(x, edge_index, W1, b1, ln_w, ln_b, W_rel, b_rel, W_root):
    f32, i32 = jnp.float32, jnp.int32
    src = edge_index[0]
    dst = edge_index[1]
    srcp = jnp.concatenate([src, jnp.zeros((EPAD - E,), i32)])
    dstp = jnp.concatenate([dst, jnp.full((EPAD - E,), N, i32)])
    srcs = jnp.concatenate([srcp, srcp + N]).reshape(2 * EROWS, CH)
    srcs4 = jnp.concatenate([srcp, srcp + N, srcp + 2 * N,
                             srcp + 3 * N]).reshape(4 * EROWS, CH)
    dst2 = dstp.reshape(EROWS, CH)
    ones16 = jnp.ones((CH, 16), i32)
    zero_h = jnp.zeros((NROWS, 16), i32)
    zero_f64 = jnp.zeros((NROWS, 64), f32)

    hist = _sc_hist(dst2, ones16, zero_h)
    hist = hist.reshape(2, NROWS, 16)[:, :N, :1]

    h2s = pl.pallas_call(
        _tc_pre,
        grid=(G,),
        in_specs=[
            pl.BlockSpec((R, D_IN), lambda g: (g, 0)),
            pl.BlockSpec((D_IN, D_HID), lambda g: (0, 0)),
            pl.BlockSpec((2, R, 1), lambda g: (0, g, 0)),
        ],
        out_specs=pl.BlockSpec((4, R, F), lambda g: (0, g, 0)),
        out_shape=jax.ShapeDtypeStruct((4, N, F), f32),
    )(x, W1, hist)

    tab1 = h2s.reshape(4 * N, F)
    agg1 = _sc_scat1(tab1, srcs4, dst2, zero_f64).reshape(4, NROWS, F)

    lastblk = lambda g: (0, jnp.minimum(g, G - 1), 0)
    outblk = lambda g: (jnp.maximum(g - G, 0), 0)
    hrs, hroot, _ = pl.pallas_call(
        _tc_mid,
        grid=(2 * G,),
        in_specs=[
            pl.BlockSpec((4, R, F), lastblk),
            pl.BlockSpec((4, R, F), lastblk),
            pl.BlockSpec((2, R, 1), lastblk),
            pl.BlockSpec((1, D_HID), lambda g: (0, 0)),
            pl.BlockSpec((1, D_HID), lambda g: (0, 0)),
            pl.BlockSpec((1, D_HID), lambda g: (0, 0)),
            pl.BlockSpec((D_HID, D_OUT), lambda g: (0, 0)),
            pl.BlockSpec((D_HID, D_OUT), lambda g: (0, 0)),
        ],
        out_specs=[
            pl.BlockSpec((2, R, 64), lambda g: (0, jnp.maximum(g - G, 0), 0)),
            pl.BlockSpec((R, D_OUT), outblk),
            pl.BlockSpec((8, 128), lambda g: (0, 0)),
        ],
        out_shape=[
            jax.ShapeDtypeStruct((2, N, 64), f32),
            jax.ShapeDtypeStruct((N, D_OUT), f32),
            jax.ShapeDtypeStruct((8, 128), f32),
        ],
        scratch_shapes=[pltpu.VMEM((N, D_HID), f32)],
    )(agg1, h2s, hist, b1.reshape(1, D_HID), ln_w.reshape(1, D_HID),
      ln_b.reshape(1, D_HID), W_rel, W_root)

    agg2 = _sc_scat2(hrs.reshape(2 * N, F), srcs, dst2,
                     zero_f64).reshape(2, NROWS, F)

    emb, logp = pl.pallas_call(
        _tc_post,
        grid=(G,),
        in_specs=[
            pl.BlockSpec((2, R, 64), lambda g: (0, g, 0)),
            pl.BlockSpec((R, D_OUT), lambda g: (g, 0)),
            pl.BlockSpec((2, R, 1), lambda g: (0, g, 0)),
            pl.BlockSpec((1, D_OUT), lambda g: (0, 0)),
        ],
        out_specs=[
            pl.BlockSpec((R, D_OUT), lambda g: (g, 0)),
            pl.BlockSpec((R, D_OUT), lambda g: (g, 0)),
        ],
        out_shape=[
            jax.ShapeDtypeStruct((N, D_OUT), f32),
            jax.ShapeDtypeStruct((N, D_OUT), f32),
        ],
    )(agg2, hroot, hist, b_rel.reshape(1, D_OUT))

    return (emb, logp)


# speedup vs baseline: 1.0296x; 1.0296x over previous
import functools

import jax
import jax.numpy as jnp
from jax import lax
from jax.experimental import pallas as pl
from jax.experimental.pallas import tpu as pltpu
from jax.experimental.pallas import tpu_sc as plsc

N = 10000
E = 160000
D_IN = 256
D_HID = 256
D_OUT = 128

NC = 2
NS = 16
CH = 128
EPAD = 163840
EROWS = EPAD // CH
NROWS = 10016
RPT = NROWS // NS
NBUF = 8
LEAD = 4

_mesh = plsc.VectorSubcoreMesh(core_axis_name="c", subcore_axis_name="s")
_params = pltpu.CompilerParams(use_tc_tiling_on_sc=False)


@functools.partial(
    pl.kernel,
    out_type=jax.ShapeDtypeStruct((2 * NROWS, 16), jnp.int32),
    mesh=_mesh,
    compiler_params=_params,
    scratch_types=[
        pltpu.VMEM((EROWS // 32, CH), jnp.int32),
        pltpu.VMEM((CH, 16), jnp.int32),
        pltpu.MemorySpace.VMEM_SHARED((NROWS, 16), jnp.int32),
        pltpu.SemaphoreType.DMA,
    ],
)
def _sc_hist(dst_hbm, ones_hbm, zero_hbm, out_hbm, dst_v, ones_v, acc, sem):
    c = lax.axis_index("c")
    s = lax.axis_index("s")
    nrow = EROWS // 32
    pltpu.sync_copy(zero_hbm.at[pl.ds(s * RPT, RPT)],
                    acc.at[pl.ds(s * RPT, RPT)])
    pltpu.sync_copy(ones_hbm, ones_v)
    pltpu.sync_copy(dst_hbm.at[pl.ds((c * NS + s) * nrow, nrow)], dst_v)
    plsc.subcore_barrier()

    def body(j, _):
        pltpu.async_copy(ones_v, acc.at[dst_v.at[j]], sem, add=True)
        return _

    lax.fori_loop(0, nrow, body, None)

    def drain(j, _):
        pltpu.make_async_copy(ones_v, acc.at[dst_v.at[j]], sem).wait()
        return _

    lax.fori_loop(0, nrow, drain, None)
    plsc.subcore_barrier()
    pltpu.sync_copy(acc.at[pl.ds(s * RPT, RPT)],
                    out_hbm.at[pl.ds(c * NROWS + s * RPT, RPT)])


F = 64


def _make_scat(nphase):
    nrow = EROWS // NS

    @functools.partial(
        pl.kernel,
        out_type=jax.ShapeDtypeStruct((2 * nphase * NROWS, F), jnp.float32),
        mesh=_mesh,
        compiler_params=_params,
        scratch_types=(
            [
                pltpu.VMEM((nrow, CH), jnp.int32),
                pltpu.VMEM((nrow, CH), jnp.int32),
                pltpu.VMEM((NBUF, CH, F), jnp.float32),
                pltpu.MemorySpace.VMEM_SHARED((NROWS, F), jnp.float32),
            ]
            + [pltpu.SemaphoreType.DMA] * (2 * NBUF)
        ),
    )
    def scat(tab_hbm, srcs_hbm, dst_hbm, zero_hbm, out_hbm,
             src_v, dst_v, rows, acc, *sems):
        gsem = sems[:NBUF]
        ssem = sems[NBUF:]
        c = lax.axis_index("c")
        s = lax.axis_index("s")
        pltpu.sync_copy(dst_hbm.at[pl.ds(s * nrow, nrow)], dst_v)
        for p in range(nphase):
            pltpu.sync_copy(
                srcs_hbm.at[pl.ds((p * 2 + c) * EROWS + s * nrow, nrow)],
                src_v)
            for b in range(LEAD):
                pltpu.async_copy(tab_hbm.at[src_v.at[b]], rows.at[b], gsem[b])
            pltpu.sync_copy(zero_hbm.at[pl.ds(s * RPT, RPT)],
                            acc.at[pl.ds(s * RPT, RPT)])
            plsc.subcore_barrier()

            def body(jj, _):
                for b in range(NBUF):
                    j = jj * NBUF + b
                    pltpu.make_async_copy(tab_hbm.at[src_v.at[j]],
                                          rows.at[b], gsem[b]).wait()
                    pltpu.async_copy(rows.at[b], acc.at[dst_v.at[j]],
                                     ssem[b], add=True)
                    pj = j + LEAD
                    pb = (b + LEAD) % NBUF

                    @pl.when(pj < nrow)
                    def _():
                        @pl.when(pj >= NBUF)
                        def _():
                            pltpu.make_async_copy(
                                rows.at[pb], acc.at[dst_v.at[pj - NBUF]],
                                ssem[pb]).wait()

                        pltpu.async_copy(tab_hbm.at[src_v.at[pj]],
                                         rows.at[pb], gsem[pb])
                return _

            lax.fori_loop(0, nrow // NBUF, body, None)
            for b in range(NBUF):
                j = nrow - NBUF + b
                pltpu.make_async_copy(rows.at[b], acc.at[dst_v.at[j]],
                                      ssem[b]).wait()
            plsc.subcore_barrier()
            pltpu.sync_copy(
                acc.at[pl.ds(s * RPT, RPT)],
                out_hbm.at[pl.ds((p * 2 + c) * NROWS + s * RPT, RPT)])
            if p + 1 < nphase:
                plsc.subcore_barrier()

    return scat


_sc_scat1 = _make_scat(2)
_sc_scat2 = _make_scat(1)


R = 1000
G = N // R


def _cnt_block(hist_ref):
    return (hist_ref[0] + hist_ref[1]).astype(jnp.float32)


def _tc_pre(x_ref, w_ref, hist_ref, out_ref):
    h = jnp.dot(x_ref[...], w_ref[...], preferred_element_type=jnp.float32)
    h = h * lax.rsqrt(_cnt_block(hist_ref) + 1.0)
    for q in range(4):
        out_ref[q] = h[:, q * F:(q + 1) * F]


def _tc_mid_a(agg_ref, h2_ref, hist_ref, b1_ref, g_ref, mom_ref):
    dinv = lax.rsqrt(_cnt_block(hist_ref) + 1.0)
    aggc = jnp.concatenate([agg_ref[0], agg_ref[1],
                            agg_ref[2], agg_ref[3]], axis=1)
    h2c = jnp.concatenate([h2_ref[0], h2_ref[1], h2_ref[2], h2_ref[3]],
                          axis=1)
    g = (aggc + h2c) * dinv + b1_ref[...]
    g_ref[...] = g
    srow = lax.broadcasted_iota(jnp.int32, (8, 128), 0)
    scol = lax.broadcasted_iota(jnp.int32, (8, 128), 1)
    contrib = (jnp.where((srow == 0) & (scol == 0), jnp.sum(g), 0.0)
               + jnp.where((srow == 0) & (scol == 1), jnp.sum(g * g), 0.0))

    @pl.when(pl.program_id(0) == 0)
    def _():
        mom_ref[...] = contrib

    @pl.when(pl.program_id(0) > 0)
    def _():
        mom_ref[...] = mom_ref[...] + contrib


def _tc_mid_b(g_ref, mom_ref, lnw_ref, lnb_ref, wrel_ref, wroot_ref,
              hrs_ref, hroot_ref):
    tot = float(N * D_HID)
    mean = mom_ref[0, 0] / tot
    var = mom_ref[0, 1] / tot - mean * mean
    std = jnp.sqrt(jnp.maximum(var, 0.0))
    hh = (g_ref[...] - mean) / (std + 1e-5) * lnw_ref[...] + lnb_ref[...]
    hh = jnp.maximum(hh, 0.0)
    hr = jnp.dot(hh, wrel_ref[...], preferred_element_type=jnp.float32)
    hroot_ref[...] = jnp.dot(hh, wroot_ref[...],
                             preferred_element_type=jnp.float32)
    hrs_ref[0] = hr[:, :64]
    hrs_ref[1] = hr[:, 64:]


def _tc_post(agg_ref, hroot_ref, hist_ref, brel_ref, emb_ref, logp_ref):
    cnt = _cnt_block(hist_ref)
    aggc = jnp.concatenate([agg_ref[0], agg_ref[1]], axis=1)
    emb = aggc / jnp.maximum(cnt, 1.0) + brel_ref[...] + hroot_ref[...]
    emb_ref[...] = emb
    sh = emb - jnp.max(emb, axis=1, keepdims=True)
    logp_ref[...] = sh - jnp.log(jnp.sum(jnp.exp(sh), axis=1, keepdims=True))


def kernel(x, edge_index, W1, b1, ln_w, ln_b, W_rel, b_rel, W_root):
    f32, i32 = jnp.float32, jnp.int32
    src = edge_index[0]
    dst = edge_index[1]
    srcp = jnp.concatenate([src, jnp.zeros((EPAD - E,), i32)])
    dstp = jnp.concatenate([dst, jnp.full((EPAD - E,), N, i32)])
    srcs = jnp.concatenate([srcp, srcp + N]).reshape(2 * EROWS, CH)
    srcs4 = jnp.concatenate([srcp, srcp + N, srcp + 2 * N,
                             srcp + 3 * N]).reshape(4 * EROWS, CH)
    dst2 = dstp.reshape(EROWS, CH)
    ones16 = jnp.ones((CH, 16), i32)
    zero_h = jnp.zeros((NROWS, 16), i32)
    zero_f64 = jnp.zeros((NROWS, 64), f32)

    hist = _sc_hist(dst2, ones16, zero_h)
    hist = hist.reshape(2, NROWS, 16)[:, :N, :1]

    h2s = pl.pallas_call(
        _tc_pre,
        grid=(G,),
        in_specs=[
            pl.BlockSpec((R, D_IN), lambda g: (g, 0)),
            pl.BlockSpec((D_IN, D_HID), lambda g: (0, 0)),
            pl.BlockSpec((2, R, 1), lambda g: (0, g, 0)),
        ],
        out_specs=pl.BlockSpec((4, R, F), lambda g: (0, g, 0)),
        out_shape=jax.ShapeDtypeStruct((4, N, F), f32),
    )(x, W1, hist)

    tab1 = h2s.reshape(4 * N, F)
    agg1 = _sc_scat1(tab1, srcs4, dst2, zero_f64).reshape(4, NROWS, F)

    g, mom = pl.pallas_call(
        _tc_mid_a,
        grid=(G,),
        in_specs=[
            pl.BlockSpec((4, R, F), lambda g: (0, g, 0)),
            pl.BlockSpec((4, R, F), lambda g: (0, g, 0)),
            pl.BlockSpec((2, R, 1), lambda g: (0, g, 0)),
            pl.BlockSpec((1, D_HID), lambda g: (0, 0)),
        ],
        out_specs=[
            pl.BlockSpec((R, D_HID), lambda g: (g, 0)),
            pl.BlockSpec((8, 128), lambda g: (0, 0)),
        ],
        out_shape=[
            jax.ShapeDtypeStruct((N, D_HID), f32),
            jax.ShapeDtypeStruct((8, 128), f32),
        ],
    )(agg1, h2s, hist, b1.reshape(1, D_HID))

    hrs, hroot = pl.pallas_call(
        _tc_mid_b,
        grid=(G,),
        in_specs=[
            pl.BlockSpec((R, D_HID), lambda g: (g, 0)),
            pl.BlockSpec(memory_space=pltpu.MemorySpace.SMEM),
            pl.BlockSpec((1, D_HID), lambda g: (0, 0)),
            pl.BlockSpec((1, D_HID), lambda g: (0, 0)),
            pl.BlockSpec((D_HID, D_OUT), lambda g: (0, 0)),
            pl.BlockSpec((D_HID, D_OUT), lambda g: (0, 0)),
        ],
        out_specs=[
            pl.BlockSpec((2, R, 64), lambda g: (0, g, 0)),
            pl.BlockSpec((R, D_OUT), lambda g: (g, 0)),
        ],
        out_shape=[
            jax.ShapeDtypeStruct((2, N, 64), f32),
            jax.ShapeDtypeStruct((N, D_OUT), f32),
        ],
    )(g, mom, ln_w.reshape(1, D_HID), ln_b.reshape(1, D_HID), W_rel, W_root)

    agg2 = _sc_scat2(hrs.reshape(2 * N, F), srcs, dst2,
                     zero_f64).reshape(2, NROWS, F)

    emb, logp = pl.pallas_call(
        _tc_post,
        grid=(G,),
        in_specs=[
            pl.BlockSpec((2, R, 64), lambda g: (0, g, 0)),
            pl.BlockSpec((R, D_OUT), lambda g: (g, 0)),
            pl.BlockSpec((2, R, 1), lambda g: (0, g, 0)),
            pl.BlockSpec((1, D_OUT), lambda g: (0, 0)),
        ],
        out_specs=[
            pl.BlockSpec((R, D_OUT), lambda g: (g, 0)),
            pl.BlockSpec((R, D_OUT), lambda g: (g, 0)),
        ],
        out_shape=[
            jax.ShapeDtypeStruct((N, D_OUT), f32),
            jax.ShapeDtypeStruct((N, D_OUT), f32),
        ],
    )(agg2, hroot, hist, b_rel.reshape(1, D_OUT))

    return (emb, logp)


# speedup vs baseline: 1.0367x; 1.0069x over previous
import functools

import jax
import jax.numpy as jnp
from jax import lax
from jax.experimental import pallas as pl
from jax.experimental.pallas import tpu as pltpu
from jax.experimental.pallas import tpu_sc as plsc

N = 10000
E = 160000
D_IN = 256
D_HID = 256
D_OUT = 128

NC = 2
NS = 16
CH = 128
EPAD = 163840
EROWS = EPAD // CH
NROWS = 10016
RPT = NROWS // NS
NBUF = 8
LEAD = 6

_mesh = plsc.VectorSubcoreMesh(core_axis_name="c", subcore_axis_name="s")
_params = pltpu.CompilerParams(use_tc_tiling_on_sc=False)


@functools.partial(
    pl.kernel,
    out_type=jax.ShapeDtypeStruct((2 * NROWS, 16), jnp.int32),
    mesh=_mesh,
    compiler_params=_params,
    scratch_types=[
        pltpu.VMEM((EROWS // 32, CH), jnp.int32),
        pltpu.VMEM((CH, 16), jnp.int32),
        pltpu.MemorySpace.VMEM_SHARED((NROWS, 16), jnp.int32),
        pltpu.SemaphoreType.DMA,
    ],
)
def _sc_hist(dst_hbm, ones_hbm, zero_hbm, out_hbm, dst_v, ones_v, acc, sem):
    c = lax.axis_index("c")
    s = lax.axis_index("s")
    nrow = EROWS // 32
    pltpu.sync_copy(zero_hbm.at[pl.ds(s * RPT, RPT)],
                    acc.at[pl.ds(s * RPT, RPT)])
    pltpu.sync_copy(ones_hbm, ones_v)
    pltpu.sync_copy(dst_hbm.at[pl.ds((c * NS + s) * nrow, nrow)], dst_v)
    plsc.subcore_barrier()

    def body(j, _):
        pltpu.async_copy(ones_v, acc.at[dst_v.at[j]], sem, add=True)
        return _

    lax.fori_loop(0, nrow, body, None)

    def drain(j, _):
        pltpu.make_async_copy(ones_v, acc.at[dst_v.at[j]], sem).wait()
        return _

    lax.fori_loop(0, nrow, drain, None)
    plsc.subcore_barrier()
    pltpu.sync_copy(acc.at[pl.ds(s * RPT, RPT)],
                    out_hbm.at[pl.ds(c * NROWS + s * RPT, RPT)])


F = 64


def _make_scat(nphase):
    nrow = EROWS // NS

    @functools.partial(
        pl.kernel,
        out_type=jax.ShapeDtypeStruct((2 * nphase * NROWS, F), jnp.float32),
        mesh=_mesh,
        compiler_params=_params,
        scratch_types=(
            [
                pltpu.VMEM((nrow, CH), jnp.int32),
                pltpu.VMEM((nrow, CH), jnp.int32),
                pltpu.VMEM((NBUF, CH, F), jnp.float32),
                pltpu.MemorySpace.VMEM_SHARED((NROWS, F), jnp.float32),
            ]
            + [pltpu.SemaphoreType.DMA] * (2 * NBUF)
        ),
    )
    def scat(tab_hbm, srcs_hbm, dst_hbm, zero_hbm, out_hbm,
             src_v, dst_v, rows, acc, *sems):
        gsem = sems[:NBUF]
        ssem = sems[NBUF:]
        c = lax.axis_index("c")
        s = lax.axis_index("s")
        pltpu.sync_copy(dst_hbm.at[pl.ds(s * nrow, nrow)], dst_v)
        for p in range(nphase):
            pltpu.sync_copy(
                srcs_hbm.at[pl.ds((p * 2 + c) * EROWS + s * nrow, nrow)],
                src_v)
            for b in range(LEAD):
                pltpu.async_copy(tab_hbm.at[src_v.at[b]], rows.at[b], gsem[b])
            pltpu.sync_copy(zero_hbm.at[pl.ds(s * RPT, RPT)],
                            acc.at[pl.ds(s * RPT, RPT)])
            plsc.subcore_barrier()

            def body(jj, _):
                for b in range(NBUF):
                    j = jj * NBUF + b
                    pltpu.make_async_copy(tab_hbm.at[src_v.at[j]],
                                          rows.at[b], gsem[b]).wait()
                    pltpu.async_copy(rows.at[b], acc.at[dst_v.at[j]],
                                     ssem[b], add=True)
                    pj = j + LEAD
                    pb = (b + LEAD) % NBUF

                    @pl.when(pj < nrow)
                    def _():
                        @pl.when(pj >= NBUF)
                        def _():
                            pltpu.make_async_copy(
                                rows.at[pb], acc.at[dst_v.at[pj - NBUF]],
                                ssem[pb]).wait()

                        pltpu.async_copy(tab_hbm.at[src_v.at[pj]],
                                         rows.at[pb], gsem[pb])
                return _

            lax.fori_loop(0, nrow // NBUF, body, None)
            for b in range(NBUF):
                j = nrow - NBUF + b
                pltpu.make_async_copy(rows.at[b], acc.at[dst_v.at[j]],
                                      ssem[b]).wait()
            plsc.subcore_barrier()
            pltpu.sync_copy(
                acc.at[pl.ds(s * RPT, RPT)],
                out_hbm.at[pl.ds((p * 2 + c) * NROWS + s * RPT, RPT)])
            if p + 1 < nphase:
                plsc.subcore_barrier()

    return scat


_sc_scat1 = _make_scat(2)
_sc_scat2 = _make_scat(1)


R = 1000
G = N // R


def _cnt_block(hist_ref):
    return (hist_ref[0] + hist_ref[1]).astype(jnp.float32)


def _tc_pre(x_ref, w_ref, hist_ref, out_ref):
    h = jnp.dot(x_ref[...], w_ref[...], preferred_element_type=jnp.float32)
    h = h * lax.rsqrt(_cnt_block(hist_ref) + 1.0)
    for q in range(4):
        out_ref[q] = h[:, q * F:(q + 1) * F]


def _tc_mid_a(agg_ref, h2_ref, hist_ref, b1_ref, g_ref, mom_ref):
    dinv = lax.rsqrt(_cnt_block(hist_ref) + 1.0)
    aggc = jnp.concatenate([agg_ref[0], agg_ref[1],
                            agg_ref[2], agg_ref[3]], axis=1)
    h2c = jnp.concatenate([h2_ref[0], h2_ref[1], h2_ref[2], h2_ref[3]],
                          axis=1)
    g = (aggc + h2c) * dinv + b1_ref[...]
    g_ref[...] = g
    srow = lax.broadcasted_iota(jnp.int32, (8, 128), 0)
    scol = lax.broadcasted_iota(jnp.int32, (8, 128), 1)
    contrib = (jnp.where((srow == 0) & (scol == 0), jnp.sum(g), 0.0)
               + jnp.where((srow == 0) & (scol == 1), jnp.sum(g * g), 0.0))

    @pl.when(pl.program_id(0) == 0)
    def _():
        mom_ref[...] = contrib

    @pl.when(pl.program_id(0) > 0)
    def _():
        mom_ref[...] = mom_ref[...] + contrib


def _tc_mid_b(g_ref, mom_ref, lnw_ref, lnb_ref, wrel_ref, wroot_ref,
              hrs_ref, hroot_ref):
    tot = float(N * D_HID)
    mean = mom_ref[0, 0] / tot
    var = mom_ref[0, 1] / tot - mean * mean
    std = jnp.sqrt(jnp.maximum(var, 0.0))
    hh = (g_ref[...] - mean) / (std + 1e-5) * lnw_ref[...] + lnb_ref[...]
    hh = jnp.maximum(hh, 0.0)
    hr = jnp.dot(hh, wrel_ref[...], preferred_element_type=jnp.float32)
    hroot_ref[...] = jnp.dot(hh, wroot_ref[...],
                             preferred_element_type=jnp.float32)
    hrs_ref[0] = hr[:, :64]
    hrs_ref[1] = hr[:, 64:]


def _tc_post(agg_ref, hroot_ref, hist_ref, brel_ref, emb_ref, logp_ref):
    cnt = _cnt_block(hist_ref)
    aggc = jnp.concatenate([agg_ref[0], agg_ref[1]], axis=1)
    emb = aggc / jnp.maximum(cnt, 1.0) + brel_ref[...] + hroot_ref[...]
    emb_ref[...] = emb
    sh = emb - jnp.max(emb, axis=1, keepdims=True)
    logp_ref[...] = sh - jnp.log(jnp.sum(jnp.exp(sh), axis=1, keepdims=True))


def kernel(x, edge_index, W1, b1, ln_w, ln_b, W_rel, b_rel, W_root):
    f32, i32 = jnp.float32, jnp.int32
    src = edge_index[0]
    dst = edge_index[1]
    srcp = jnp.concatenate([src, jnp.zeros((EPAD - E,), i32)])
    dstp = jnp.concatenate([dst, jnp.full((EPAD - E,), N, i32)])
    srcs = jnp.concatenate([srcp, srcp + N]).reshape(2 * EROWS, CH)
    srcs4 = jnp.concatenate([srcp, srcp + N, srcp + 2 * N,
                             srcp + 3 * N]).reshape(4 * EROWS, CH)
    dst2 = dstp.reshape(EROWS, CH)
    ones16 = jnp.ones((CH, 16), i32)
    zero_h = jnp.zeros((NROWS, 16), i32)
    zero_f64 = jnp.zeros((NROWS, 64), f32)

    hist = _sc_hist(dst2, ones16, zero_h)
    hist = hist.reshape(2, NROWS, 16)[:, :N, :1]

    h2s = pl.pallas_call(
        _tc_pre,
        grid=(G,),
        in_specs=[
            pl.BlockSpec((R, D_IN), lambda g: (g, 0)),
            pl.BlockSpec((D_IN, D_HID), lambda g: (0, 0)),
            pl.BlockSpec((2, R, 1), lambda g: (0, g, 0)),
        ],
        out_specs=pl.BlockSpec((4, R, F), lambda g: (0, g, 0)),
        out_shape=jax.ShapeDtypeStruct((4, N, F), f32),
    )(x, W1, hist)

    tab1 = h2s.reshape(4 * N, F)
    agg1 = _sc_scat1(tab1, srcs4, dst2, zero_f64).reshape(4, NROWS, F)

    g, mom = pl.pallas_call(
        _tc_mid_a,
        grid=(G,),
        in_specs=[
            pl.BlockSpec((4, R, F), lambda g: (0, g, 0)),
            pl.BlockSpec((4, R, F), lambda g: (0, g, 0)),
            pl.BlockSpec((2, R, 1), lambda g: (0, g, 0)),
            pl.BlockSpec((1, D_HID), lambda g: (0, 0)),
        ],
        out_specs=[
            pl.BlockSpec((R, D_HID), lambda g: (g, 0)),
            pl.BlockSpec((8, 128), lambda g: (0, 0)),
        ],
        out_shape=[
            jax.ShapeDtypeStruct((N, D_HID), f32),
            jax.ShapeDtypeStruct((8, 128), f32),
        ],
    )(agg1, h2s, hist, b1.reshape(1, D_HID))

    hrs, hroot = pl.pallas_call(
        _tc_mid_b,
        grid=(G,),
        in_specs=[
            pl.BlockSpec((R, D_HID), lambda g: (g, 0)),
            pl.BlockSpec(memory_space=pltpu.MemorySpace.SMEM),
            pl.BlockSpec((1, D_HID), lambda g: (0, 0)),
            pl.BlockSpec((1, D_HID), lambda g: (0, 0)),
            pl.BlockSpec((D_HID, D_OUT), lambda g: (0, 0)),
            pl.BlockSpec((D_HID, D_OUT), lambda g: (0, 0)),
        ],
        out_specs=[
            pl.BlockSpec((2, R, 64), lambda g: (0, g, 0)),
            pl.BlockSpec((R, D_OUT), lambda g: (g, 0)),
        ],
        out_shape=[
            jax.ShapeDtypeStruct((2, N, 64), f32),
            jax.ShapeDtypeStruct((N, D_OUT), f32),
        ],
    )(g, mom, ln_w.reshape(1, D_HID), ln_b.reshape(1, D_HID), W_rel, W_root)

    agg2 = _sc_scat2(hrs.reshape(2 * N, F), srcs, dst2,
                     zero_f64).reshape(2, NROWS, F)

    emb, logp = pl.pallas_call(
        _tc_post,
        grid=(G,),
        in_specs=[
            pl.BlockSpec((2, R, 64), lambda g: (0, g, 0)),
            pl.BlockSpec((R, D_OUT), lambda g: (g, 0)),
            pl.BlockSpec((2, R, 1), lambda g: (0, g, 0)),
            pl.BlockSpec((1, D_OUT), lambda g: (0, 0)),
        ],
        out_specs=[
            pl.BlockSpec((R, D_OUT), lambda g: (g, 0)),
            pl.BlockSpec((R, D_OUT), lambda g: (g, 0)),
        ],
        out_shape=[
            jax.ShapeDtypeStruct((N, D_OUT), f32),
            jax.ShapeDtypeStruct((N, D_OUT), f32),
        ],
    )(agg2, hroot, hist, b_rel.reshape(1, D_OUT))

    return (emb, logp)


# speedup vs baseline: 1.0548x; 1.0175x over previous
import functools

import jax
import jax.numpy as jnp
from jax import lax
from jax.experimental import pallas as pl
from jax.experimental.pallas import tpu as pltpu
from jax.experimental.pallas import tpu_sc as plsc

N = 10000
E = 160000
D_IN = 256
D_HID = 256
D_OUT = 128

NC = 2
NS = 16
CH = 128
EPAD = 163840
EROWS = EPAD // CH
NROWS = 10016
RPT = NROWS // NS
NBUF = 8
LEAD = 6

_mesh = plsc.VectorSubcoreMesh(core_axis_name="c", subcore_axis_name="s")
_params = pltpu.CompilerParams(use_tc_tiling_on_sc=False)


@functools.partial(
    pl.kernel,
    out_type=jax.ShapeDtypeStruct((2 * NROWS, 16), jnp.int32),
    mesh=_mesh,
    compiler_params=_params,
    scratch_types=[
        pltpu.VMEM((EROWS // 32, CH), jnp.int32),
        pltpu.VMEM((CH, 16), jnp.int32),
        pltpu.MemorySpace.VMEM_SHARED((NROWS, 16), jnp.int32),
        pltpu.SemaphoreType.DMA,
    ],
)
def _sc_hist(dst_hbm, ones_hbm, zero_hbm, out_hbm, dst_v, ones_v, acc, sem):
    c = lax.axis_index("c")
    s = lax.axis_index("s")
    nrow = EROWS // 32
    pltpu.sync_copy(zero_hbm.at[pl.ds(s * RPT, RPT)],
                    acc.at[pl.ds(s * RPT, RPT)])
    pltpu.sync_copy(ones_hbm, ones_v)
    pltpu.sync_copy(dst_hbm.at[pl.ds((c * NS + s) * nrow, nrow)], dst_v)
    plsc.subcore_barrier()

    def body(j, _):
        pltpu.async_copy(ones_v, acc.at[dst_v.at[j]], sem, add=True)
        return _

    lax.fori_loop(0, nrow, body, None)

    def drain(j, _):
        pltpu.make_async_copy(ones_v, acc.at[dst_v.at[j]], sem).wait()
        return _

    lax.fori_loop(0, nrow, drain, None)
    plsc.subcore_barrier()
    pltpu.sync_copy(acc.at[pl.ds(s * RPT, RPT)],
                    out_hbm.at[pl.ds(c * NROWS + s * RPT, RPT)])


F = 64


def _make_scat(nphase):
    nrow = EROWS // NS

    @functools.partial(
        pl.kernel,
        out_type=jax.ShapeDtypeStruct((2 * nphase * NROWS, F), jnp.float32),
        mesh=_mesh,
        compiler_params=_params,
        scratch_types=(
            [
                pltpu.VMEM((nrow, CH), jnp.int32),
                pltpu.VMEM((nrow, CH), jnp.int32),
                pltpu.VMEM((NBUF, CH, F), jnp.float32),
                pltpu.MemorySpace.VMEM_SHARED((NROWS, F), jnp.float32),
            ]
            + [pltpu.SemaphoreType.DMA] * (3 * NBUF)
        ),
    )
    def scat(tab_hbm, srcs_hbm, dst_hbm, zero_hbm, out_hbm,
             src_v, dst_v, rows, acc, *sems):
        gsem = sems[:NBUF]
        gsem2 = sems[NBUF:2 * NBUF]
        ssem = sems[2 * NBUF:]

        def gather(j, b):
            pltpu.async_copy(tab_hbm.at[src_v.at[j, pl.ds(0, 64)]],
                             rows.at[b, pl.ds(0, 64)], gsem[b])
            pltpu.async_copy(tab_hbm.at[src_v.at[j, pl.ds(64, 64)]],
                             rows.at[b, pl.ds(64, 64)], gsem2[b])

        def gather_wait(j, b):
            pltpu.make_async_copy(tab_hbm.at[src_v.at[j, pl.ds(0, 64)]],
                                  rows.at[b, pl.ds(0, 64)], gsem[b]).wait()
            pltpu.make_async_copy(tab_hbm.at[src_v.at[j, pl.ds(64, 64)]],
                                  rows.at[b, pl.ds(64, 64)], gsem2[b]).wait()
        c = lax.axis_index("c")
        s = lax.axis_index("s")
        pltpu.sync_copy(dst_hbm.at[pl.ds(s * nrow, nrow)], dst_v)
        for p in range(nphase):
            pltpu.sync_copy(
                srcs_hbm.at[pl.ds((p * 2 + c) * EROWS + s * nrow, nrow)],
                src_v)
            for b in range(LEAD):
                gather(b, b)
            pltpu.sync_copy(zero_hbm.at[pl.ds(s * RPT, RPT)],
                            acc.at[pl.ds(s * RPT, RPT)])
            plsc.subcore_barrier()

            def body(jj, _):
                for b in range(NBUF):
                    j = jj * NBUF + b
                    gather_wait(j, b)
                    pltpu.async_copy(rows.at[b], acc.at[dst_v.at[j]],
                                     ssem[b], add=True)
                    pj = j + LEAD
                    pb = (b + LEAD) % NBUF

                    @pl.when(pj < nrow)
                    def _():
                        @pl.when(pj >= NBUF)
                        def _():
                            pltpu.make_async_copy(
                                rows.at[pb], acc.at[dst_v.at[pj - NBUF]],
                                ssem[pb]).wait()

                        gather(pj, pb)
                return _

            lax.fori_loop(0, nrow // NBUF, body, None)
            for b in range(NBUF):
                j = nrow - NBUF + b
                pltpu.make_async_copy(rows.at[b], acc.at[dst_v.at[j]],
                                      ssem[b]).wait()
            plsc.subcore_barrier()
            pltpu.sync_copy(
                acc.at[pl.ds(s * RPT, RPT)],
                out_hbm.at[pl.ds((p * 2 + c) * NROWS + s * RPT, RPT)])
            if p + 1 < nphase:
                plsc.subcore_barrier()

    return scat


_sc_scat1 = _make_scat(2)
_sc_scat2 = _make_scat(1)


R = 1000
G = N // R


def _cnt_block(hist_ref):
    return (hist_ref[0] + hist_ref[1]).astype(jnp.float32)


def _tc_pre(x_ref, w_ref, hist_ref, out_ref):
    h = jnp.dot(x_ref[...], w_ref[...], preferred_element_type=jnp.float32)
    h = h * lax.rsqrt(_cnt_block(hist_ref) + 1.0)
    for q in range(4):
        out_ref[q] = h[:, q * F:(q + 1) * F]


def _tc_mid_a(agg_ref, h2_ref, hist_ref, b1_ref, g_ref, mom_ref):
    dinv = lax.rsqrt(_cnt_block(hist_ref) + 1.0)
    aggc = jnp.concatenate([agg_ref[0], agg_ref[1],
                            agg_ref[2], agg_ref[3]], axis=1)
    h2c = jnp.concatenate([h2_ref[0], h2_ref[1], h2_ref[2], h2_ref[3]],
                          axis=1)
    g = (aggc + h2c) * dinv + b1_ref[...]
    g_ref[...] = g
    srow = lax.broadcasted_iota(jnp.int32, (8, 128), 0)
    scol = lax.broadcasted_iota(jnp.int32, (8, 128), 1)
    contrib = (jnp.where((srow == 0) & (scol == 0), jnp.sum(g), 0.0)
               + jnp.where((srow == 0) & (scol == 1), jnp.sum(g * g), 0.0))

    @pl.when(pl.program_id(0) == 0)
    def _():
        mom_ref[...] = contrib

    @pl.when(pl.program_id(0) > 0)
    def _():
        mom_ref[...] = mom_ref[...] + contrib


def _tc_mid_b(g_ref, mom_ref, lnw_ref, lnb_ref, wrel_ref, wroot_ref,
              hrs_ref, hroot_ref):
    tot = float(N * D_HID)
    mean = mom_ref[0, 0] / tot
    var = mom_ref[0, 1] / tot - mean * mean
    std = jnp.sqrt(jnp.maximum(var, 0.0))
    hh = (g_ref[...] - mean) / (std + 1e-5) * lnw_ref[...] + lnb_ref[...]
    hh = jnp.maximum(hh, 0.0)
    hr = jnp.dot(hh, wrel_ref[...], preferred_element_type=jnp.float32)
    hroot_ref[...] = jnp.dot(hh, wroot_ref[...],
                             preferred_element_type=jnp.float32)
    hrs_ref[0] = hr[:, :64]
    hrs_ref[1] = hr[:, 64:]


def _tc_post(agg_ref, hroot_ref, hist_ref, brel_ref, emb_ref, logp_ref):
    cnt = _cnt_block(hist_ref)
    aggc = jnp.concatenate([agg_ref[0], agg_ref[1]], axis=1)
    emb = aggc / jnp.maximum(cnt, 1.0) + brel_ref[...] + hroot_ref[...]
    emb_ref[...] = emb
    sh = emb - jnp.max(emb, axis=1, keepdims=True)
    logp_ref[...] = sh - jnp.log(jnp.sum(jnp.exp(sh), axis=1, keepdims=True))


def kernel(x, edge_index, W1, b1, ln_w, ln_b, W_rel, b_rel, W_root):
    f32, i32 = jnp.float32, jnp.int32
    src = edge_index[0]
    dst = edge_index[1]
    srcp = jnp.concatenate([src, jnp.zeros((EPAD - E,), i32)])
    dstp = jnp.concatenate([dst, jnp.full((EPAD - E,), N, i32)])
    srcs = jnp.concatenate([srcp, srcp + N]).reshape(2 * EROWS, CH)
    srcs4 = jnp.concatenate([srcp, srcp + N, srcp + 2 * N,
                             srcp + 3 * N]).reshape(4 * EROWS, CH)
    dst2 = dstp.reshape(EROWS, CH)
    ones16 = jnp.ones((CH, 16), i32)
    zero_h = jnp.zeros((NROWS, 16), i32)
    zero_f64 = jnp.zeros((NROWS, 64), f32)

    hist = _sc_hist(dst2, ones16, zero_h)
    hist = hist.reshape(2, NROWS, 16)[:, :N, :1]

    h2s = pl.pallas_call(
        _tc_pre,
        grid=(G,),
        in_specs=[
            pl.BlockSpec((R, D_IN), lambda g: (g, 0)),
            pl.BlockSpec((D_IN, D_HID), lambda g: (0, 0)),
            pl.BlockSpec((2, R, 1), lambda g: (0, g, 0)),
        ],
        out_specs=pl.BlockSpec((4, R, F), lambda g: (0, g, 0)),
        out_shape=jax.ShapeDtypeStruct((4, N, F), f32),
    )(x, W1, hist)

    tab1 = h2s.reshape(4 * N, F)
    agg1 = _sc_scat1(tab1, srcs4, dst2, zero_f64).reshape(4, NROWS, F)

    g, mom = pl.pallas_call(
        _tc_mid_a,
        grid=(G,),
        in_specs=[
            pl.BlockSpec((4, R, F), lambda g: (0, g, 0)),
            pl.BlockSpec((4, R, F), lambda g: (0, g, 0)),
            pl.BlockSpec((2, R, 1), lambda g: (0, g, 0)),
            pl.BlockSpec((1, D_HID), lambda g: (0, 0)),
        ],
        out_specs=[
            pl.BlockSpec((R, D_HID), lambda g: (g, 0)),
            pl.BlockSpec((8, 128), lambda g: (0, 0)),
        ],
        out_shape=[
            jax.ShapeDtypeStruct((N, D_HID), f32),
            jax.ShapeDtypeStruct((8, 128), f32),
        ],
    )(agg1, h2s, hist, b1.reshape(1, D_HID))

    hrs, hroot = pl.pallas_call(
        _tc_mid_b,
        grid=(G,),
        in_specs=[
            pl.BlockSpec((R, D_HID), lambda g: (g, 0)),
            pl.BlockSpec(memory_space=pltpu.MemorySpace.SMEM),
            pl.BlockSpec((1, D_HID), lambda g: (0, 0)),
            pl.BlockSpec((1, D_HID), lambda g: (0, 0)),
            pl.BlockSpec((D_HID, D_OUT), lambda g: (0, 0)),
            pl.BlockSpec((D_HID, D_OUT), lambda g: (0, 0)),
        ],
        out_specs=[
            pl.BlockSpec((2, R, 64), lambda g: (0, g, 0)),
            pl.BlockSpec((R, D_OUT), lambda g: (g, 0)),
        ],
        out_shape=[
            jax.ShapeDtypeStruct((2, N, 64), f32),
            jax.ShapeDtypeStruct((N, D_OUT), f32),
        ],
    )(g, mom, ln_w.reshape(1, D_HID), ln_b.reshape(1, D_HID), W_rel, W_root)

    agg2 = _sc_scat2(hrs.reshape(2 * N, F), srcs, dst2,
                     zero_f64).reshape(2, NROWS, F)

    emb, logp = pl.pallas_call(
        _tc_post,
        grid=(G,),
        in_specs=[
            pl.BlockSpec((2, R, 64), lambda g: (0, g, 0)),
            pl.BlockSpec((R, D_OUT), lambda g: (g, 0)),
            pl.BlockSpec((2, R, 1), lambda g: (0, g, 0)),
            pl.BlockSpec((1, D_OUT), lambda g: (0, 0)),
        ],
        out_specs=[
            pl.BlockSpec((R, D_OUT), lambda g: (g, 0)),
            pl.BlockSpec((R, D_OUT), lambda g: (g, 0)),
        ],
        out_shape=[
            jax.ShapeDtypeStruct((N, D_OUT), f32),
            jax.ShapeDtypeStruct((N, D_OUT), f32),
        ],
    )(agg2, hroot, hist, b_rel.reshape(1, D_OUT))

    return (emb, logp)
